# Initial kernel scaffold; baseline (speedup 1.0000x reference)
import jax
import jax.numpy as jnp
from jax.experimental import pallas as pl


def kernel(xyz):
    raise NotImplementedError("write your pallas kernel here")


# speedup vs baseline: 1.2561x; 1.2561x over previous
import functools

import jax
import jax.numpy as jnp
import numpy as np
from jax.experimental import pallas as pl
from jax.experimental.pallas import tpu as pltpu

NG = 128
GS = 32
KNN = 16
MIN_PTS = 4
CP = 256
HI = jax.lax.Precision.HIGHEST


def _fiota(shape, dim):
    return jax.lax.broadcasted_iota(jnp.int32, shape, dim).astype(jnp.float32)


def _knn_kernel(rows_ref, full_ref, br_ref, bT_ref, x2c_ref, x2r_ref,
                nn_ref, cov_ref):
    R = rows_ref.shape[1]
    N = full_ref.shape[1]
    xr = rows_ref[0]
    xfull = full_ref[0]
    ar = br_ref[0]
    bT = bT_ref[0]
    x2c = x2c_ref[0]
    x2r = x2r_ref[0]

    dot = jax.lax.dot_general(ar, bT, (((1,), (0,)), ((), ())),
                              preferred_element_type=jnp.float32)
    d2 = jnp.maximum(x2c + x2r - 2.0 * dot, 0.0)

    iota = _fiota((R, N), 1)

    def step(k, carry):
        d2, nn_val, acc = carry
        m = jnp.min(d2, axis=1, keepdims=True)
        eqm = d2 == m
        fi = jnp.min(jnp.where(eqm, iota, float(N)), axis=1, keepdims=True)
        onehot_b = iota == fi
        nn_val = jnp.where(k == 1, m, nn_val)
        oh = onehot_b.astype(jnp.float32)
        nbr = jax.lax.dot_general(oh, xfull, (((1,), (0,)), ((), ())),
                                  precision=HI,
                                  preferred_element_type=jnp.float32)
        ce = (nbr - xr).astype(jnp.bfloat16).astype(jnp.float32)
        c = [ce[:, 0:1], ce[:, 1:2], ce[:, 2:3]]
        contrib = jnp.concatenate(
            [c[i] * c[j] for i in range(3) for j in range(3)], axis=1)
        acc = jnp.where(k >= 1, acc + contrib, acc)
        d2 = jnp.where(onehot_b, jnp.inf, d2)
        return d2, nn_val, acc

    init = (d2, jnp.zeros((R, 1), jnp.float32), jnp.zeros((R, 9), jnp.float32))
    _, nn_val, acc = jax.lax.fori_loop(0, KNN + 1, step, init)
    nn_ref[0] = nn_val
    cov_ref[0] = acc


def _knn_pallas(xyz):
    B, N, _ = xyz.shape
    R = 128
    T = N // R
    xb = xyz.astype(jnp.bfloat16)
    xbT = jnp.transpose(xb, (0, 2, 1))
    x2 = jnp.sum(xyz * xyz, axis=-1)
    x2c = x2[:, :, None]
    x2r = x2[:, None, :]
    nn, cov9 = pl.pallas_call(
        _knn_kernel,
        grid=(B, T),
        in_specs=[
            pl.BlockSpec((1, R, 3), lambda b, t: (b, t, 0)),
            pl.BlockSpec((1, N, 3), lambda b, t: (b, 0, 0)),
            pl.BlockSpec((1, R, 3), lambda b, t: (b, t, 0)),
            pl.BlockSpec((1, 3, N), lambda b, t: (b, 0, 0)),
            pl.BlockSpec((1, R, 1), lambda b, t: (b, t, 0)),
            pl.BlockSpec((1, 1, N), lambda b, t: (b, 0, 0)),
        ],
        out_specs=[
            pl.BlockSpec((1, R, 1), lambda b, t: (b, t, 0)),
            pl.BlockSpec((1, R, 9), lambda b, t: (b, t, 0)),
        ],
        out_shape=[
            jax.ShapeDtypeStruct((B, N, 1), jnp.float32),
            jax.ShapeDtypeStruct((B, N, 9), jnp.float32),
        ],
    )(xyz, xyz, xb, xbT, x2c, x2r)
    return nn[:, :, 0], cov9


def _octree_kernel(scal_ref, gcr_ref, gcc_ref, nrm_ref, pts_ref,
                   nb_ref, ct_ref):
    b = pl.program_id(0)
    N = gcr_ref.shape[2]
    Df = scal_ref[b, 0]
    sinv1 = scal_ref[b, 1]

    gxr = gcr_ref[0, 0:1, :]
    gyr = gcr_ref[0, 1:2, :]
    gzr = gcr_ref[0, 2:3, :]
    gxc = gcc_ref[0, :, 0:1]
    gyc = gcc_ref[0, :, 1:2]
    gzc = gcc_ref[0, :, 2:3]
    nrm = nrm_ref[0]
    pcol = pts_ref[0]
    feat = jnp.concatenate([nrm, jnp.ones((N, 1), jnp.float32)], axis=1)

    slots = _fiota((1, CP), 1)
    c8col = _fiota((8, 1), 0)
    c8row = _fiota((1, 8), 1)
    bx8 = jnp.floor(c8col * 0.25)
    by8 = jnp.floor(c8col * 0.5) - 2.0 * bx8
    bz8 = c8col - 2.0 * jnp.floor(c8col * 0.5)
    i8 = _fiota((8, 8), 0)
    j8 = _fiota((8, 8), 1)
    tri8 = (j8 < i8).astype(jnp.float32)

    def child_stats(pm_row, pm_col, sinv_c):
        def bits(gr):
            return jnp.floor(gr * sinv_c) - 2.0 * jnp.floor(gr * (0.5 * sinv_c))
        ccr = 4.0 * bits(gxr) + 2.0 * bits(gyr) + bits(gzr)
        ccc = 4.0 * bits(gxc) + 2.0 * bits(gyc) + bits(gzc)
        oh = jnp.where((ccr == c8col) & pm_row, 1.0, 0.0)
        ohT = jnp.where((ccc == c8row) & pm_col, 1.0, 0.0)
        sums = jax.lax.dot_general(oh, feat, (((1,), (0,)), ((), ())),
                                   precision=HI,
                                   preferred_element_type=jnp.float32)
        cnt = sums[:, 3:4]
        safe = jnp.maximum(cnt, 1.0)
        mean = sums[:, 0:3] / safe
        meanpt = jax.lax.dot_general(ohT, mean, (((1,), (0,)), ((), ())),
                                     precision=HI,
                                     preferred_element_type=jnp.float32)
        diff = nrm - meanpt
        sq = (diff[:, 0:1] * diff[:, 0:1] + diff[:, 1:2] * diff[:, 1:2]) \
            + diff[:, 2:3] * diff[:, 2:3]
        vsum = jax.lax.dot_general(oh, sq, (((1,), (0,)), ((), ())),
                                   precision=HI,
                                   preferred_element_type=jnp.float32)
        var = jnp.where(cnt >= float(MIN_PTS), vsum / safe, 0.0)
        return cnt, var

    def insert(state_sh, vals, pos, validf):
        match = jnp.where((slots == pos) & (validf > 0.0), 1.0, 0.0)
        add = jnp.sum(vals * match, axis=0, keepdims=True)
        anym = jnp.max(match, axis=0, keepdims=True)
        return state_sh * (1.0 - anym) + add

    ones_row = jnp.ones((1, N), jnp.bool_)
    ones_col = jnp.ones((N, 1), jnp.bool_)
    cnt0, var0 = child_stats(ones_row, ones_col, sinv1)
    valid0 = (cnt0 >= 1.0).astype(jnp.float32)
    spl0 = valid0 * ((1.0 < Df) & (cnt0 >= float(MIN_PTS))).astype(jnp.float32)
    rank0 = jax.lax.dot_general(tri8, valid0, (((1,), (0,)), ((), ())),
                                precision=HI, preferred_element_type=jnp.float32)
    n0 = jnp.sum(valid0)
    zero = jnp.zeros((1, CP), jnp.float32)
    kx = insert(zero, bx8, rank0, valid0)
    ky = insert(zero, by8, rank0, valid0)
    kz = insert(zero, bz8, rank0, valid0)
    dd = insert(jnp.full((1, CP), 1.0), jnp.ones((8, 1), jnp.float32), rank0, valid0)
    var = insert(zero, var0, rank0, valid0)
    cnt = insert(zero, cnt0, rank0, valid0)
    spl = insert(zero, spl0, rank0, valid0)
    sinv = insert(jnp.full((1, CP), sinv1), jnp.full((8, 1), sinv1), rank0, valid0)
    cont0 = (n0 < float(NG)) & (jnp.max(jnp.where(slots < n0, spl, 0.0)) > 0.0)

    def cond(carry):
        return carry[-1]

    def body(carry):
        kx, ky, kz, dd, var, cnt, spl, sinv, n, _ = carry
        active = slots < n
        scores = jnp.where(active & (spl > 0.0), var, -jnp.inf)
        smax = jnp.max(scores)
        si = jnp.min(jnp.where(scores == smax, slots, float(CP)))
        sel1 = jnp.where(slots == si, 1.0, 0.0)
        pd = jnp.sum(sel1 * dd)
        kxp = jnp.sum(sel1 * kx)
        kyp = jnp.sum(sel1 * ky)
        kzp = jnp.sum(sel1 * kz)
        sinv_p = jnp.sum(sel1 * sinv)

        pm_row = (jnp.floor(gxr * sinv_p) == kxp) \
            & (jnp.floor(gyr * sinv_p) == kyp) \
            & (jnp.floor(gzr * sinv_p) == kzp)
        pm_col = (jnp.floor(gxc * sinv_p) == kxp) \
            & (jnp.floor(gyc * sinv_p) == kyp) \
            & (jnp.floor(gzc * sinv_p) == kzp)
        sinv_c = 2.0 * sinv_p
        cnt_c, var_c = child_stats(pm_row, pm_col, sinv_c)
        ckx = 2.0 * kxp + bx8
        cky = 2.0 * kyp + by8
        ckz = 2.0 * kzp + bz8
        cdd = pd + 1.0
        validf = (cnt_c >= 1.0).astype(jnp.float32)
        csplf = validf * ((cdd < Df) & (cnt_c >= float(MIN_PTS))).astype(jnp.float32)
        rankv = jax.lax.dot_general(tri8, validf, (((1,), (0,)), ((), ())),
                                    precision=HI,
                                    preferred_element_type=jnp.float32)
        nch = jnp.sum(validf)
        keep = n - 1.0
        pos = keep + rankv

        def shift(S):
            return jnp.where(slots < si, S,
                             jnp.concatenate([S[:, 1:], S[:, -1:]], axis=1))

        e8 = jnp.ones((8, 1), jnp.float32)
        kx = insert(shift(kx), ckx, pos, validf)
        ky = insert(shift(ky), cky, pos, validf)
        kz = insert(shift(kz), ckz, pos, validf)
        dd = insert(shift(dd), cdd * e8, pos, validf)
        var = insert(shift(var), var_c, pos, validf)
        cnt = insert(shift(cnt), cnt_c, pos, validf)
        spl = insert(shift(spl), csplf, pos, validf)
        sinv = insert(shift(sinv), sinv_c * e8, pos, validf)
        n = keep + nch
        cont = (n < float(NG)) & (jnp.max(jnp.where(slots < n, spl, 0.0)) > 0.0)
        return kx, ky, kz, dd, var, cnt, spl, sinv, n, cont

    carry = jax.lax.while_loop(
        cond, body, (kx, ky, kz, dd, var, cnt, spl, sinv, n0, cont0))
    kx, ky, kz, dd, var, cnt, spl, sinv, n, _ = carry

    active = slots < n
    iCP = _fiota((CP, CP), 0)
    jCP = _fiota((CP, CP), 1)

    def stable_order(key_row):
        kc = jnp.sum(jnp.where(iCP == jCP, key_row, 0.0), axis=1,
                     keepdims=True)
        cmp = (key_row < kc) | ((key_row == kc) & (jCP < iCP))
        rank = jnp.sum(cmp.astype(jnp.float32), axis=1, keepdims=True)
        onehot = (rank == jCP).astype(jnp.float32)
        icol = _fiota((CP, 1), 0)
        return jnp.sum(icol * onehot, axis=0, keepdims=True)

    inf = jnp.inf
    orda = stable_order(jnp.where(active, var, inf))
    ordd = stable_order(jnp.where(active, -var, inf))

    j128 = _fiota((NG, 1), 0)
    p128 = _fiota((NG, CP), 1)

    def gather_row(row, idx):
        m = (p128 == idx).astype(jnp.float32)
        return jnp.sum(m * row, axis=1, keepdims=True)

    start = jnp.clip(n - float(NG), 0.0, float(CP - NG))
    start = jnp.minimum(start, 8.0)
    selA = gather_row(orda, start + j128)
    selB = j128
    n_safe = jnp.maximum(n, 1.0)
    jm = j128 - n
    jw = jnp.where(j128 < n, j128, jm - n_safe * jnp.floor(jm / n_safe))
    selC = gather_row(ordd, jw)
    sel = jnp.where(n > float(NG), selA, jnp.where(n == float(NG), selB, selC))

    kxg = gather_row(kx, sel)
    kyg = gather_row(ky, sel)
    kzg = gather_row(kz, sel)
    cntg = gather_row(cnt, sel)
    sinvg = gather_row(sinv, sel)

    maskb = (jnp.floor(gxr * sinvg) == kxg) \
        & (jnp.floor(gyr * sinvg) == kyg) \
        & (jnp.floor(gzr * sinvg) == kzg)
    maskf = maskb.astype(jnp.float32)

    ctr = jax.lax.dot_general(maskf, pcol, (((1,), (0,)), ((), ())),
                              precision=HI,
                              preferred_element_type=jnp.float32) / cntg

    r = maskf
    s = 1
    while s < N:
        z = jnp.zeros((NG, s), jnp.float32)
        r = r + jnp.concatenate([z, r[:, :-s]], axis=1)
        s *= 2
    rank0b = r - 1.0

    pieces = []
    scale = jnp.full((NG, 1), -jnp.inf)
    for t in range(GS):
        tf = float(t)
        tmod = tf - cntg * jnp.floor(tf / cntg)
        hit = ((rank0b == tmod) & maskb).astype(jnp.float32)
        pt = jax.lax.dot_general(hit, pcol, (((1,), (0,)), ((), ())),
                                 precision=HI,
                                 preferred_element_type=jnp.float32)
        nbt = pt - ctr
        nsq = (nbt[:, 0:1] * nbt[:, 0:1] + nbt[:, 1:2] * nbt[:, 1:2]) \
            + nbt[:, 2:3] * nbt[:, 2:3]
        scale = jnp.maximum(scale, jnp.sqrt(nsq + 1e-12))
        pieces.append(nbt)
    scale = jnp.maximum(scale, 1e-6)
    nb_ref[0] = jnp.concatenate(pieces, axis=1) / scale
    ct_ref[0] = ctr


def _octree_pallas(scal, gcr, gcc, nrm, pts):
    B, _, N = gcr.shape
    grid_spec = pltpu.PrefetchScalarGridSpec(
        num_scalar_prefetch=1,
        grid=(B,),
        in_specs=[
            pl.BlockSpec((1, 3, N), lambda b, s: (b, 0, 0)),
            pl.BlockSpec((1, N, 3), lambda b, s: (b, 0, 0)),
            pl.BlockSpec((1, N, 3), lambda b, s: (b, 0, 0)),
            pl.BlockSpec((1, N, 3), lambda b, s: (b, 0, 0)),
        ],
        out_specs=[
            pl.BlockSpec((1, NG, 3 * GS), lambda b, s: (b, 0, 0)),
            pl.BlockSpec((1, NG, 3), lambda b, s: (b, 0, 0)),
        ],
    )
    nb, ct = pl.pallas_call(
        _octree_kernel,
        grid_spec=grid_spec,
        out_shape=[
            jax.ShapeDtypeStruct((B, NG, 3 * GS), jnp.float32),
            jax.ShapeDtypeStruct((B, NG, 3), jnp.float32),
        ],
    )(scal, gcr, gcc, nrm, pts)
    return nb, ct


_POW2 = np.ldexp(1.0, 1 - np.arange(32)).astype(np.float32)


def kernel(xyz):
    B, N, _ = xyz.shape
    nn_d2, cov9 = _knn_pallas(xyz)
    cov = cov9.reshape(B, N, 3, 3) / float(KNN)
    _, eigvecs = jnp.linalg.eigh(cov)
    normals = eigvecs[..., :, 0]
    nn_dists = jnp.sqrt(jnp.maximum(nn_d2, 1e-08))

    grid_size = jnp.maximum(jnp.quantile(nn_dists, 0.5, axis=1), 1e-06)
    xyz_min = xyz.min(axis=1)
    xyz_max = xyz.max(axis=1)
    bbox_diag = jnp.linalg.norm(xyz_max - xyz_min, axis=-1)
    raw_depth = jnp.log2(jnp.maximum(bbox_diag / grid_size, 1.0))
    depth = jnp.clip(jnp.ceil(raw_depth).astype(jnp.int32), 8, 16)
    gcf = jnp.floor((xyz - xyz_min[:, None, :]) / grid_size[:, None, None])

    scal = jnp.stack([depth.astype(jnp.float32),
                      jnp.take(jnp.asarray(_POW2), depth)], axis=1)
    gcr = jnp.transpose(gcf, (0, 2, 1))
    nb, ct = _octree_pallas(scal, gcr, gcf, normals, xyz)
    return nb.reshape(B, NG, GS, 3), ct


# speedup vs baseline: 1.3303x; 1.0590x over previous
import functools

import jax
import jax.numpy as jnp
import numpy as np
from jax.experimental import pallas as pl
from jax.experimental.pallas import tpu as pltpu

NG = 128
GS = 32
KNN = 16
MIN_PTS = 4
CP = 256
HI = jax.lax.Precision.HIGHEST


def _fiota(shape, dim):
    return jax.lax.broadcasted_iota(jnp.int32, shape, dim).astype(jnp.float32)


def _knn_kernel(rows_ref, br_ref, bT_ref, x2c_ref, x2r_ref, xT_ref,
                nn_ref, cov_ref):
    R = rows_ref.shape[1]
    N = bT_ref.shape[2]
    xr = rows_ref[0]
    ar = br_ref[0]
    bT = bT_ref[0]
    x2c = x2c_ref[0]
    x2r = x2r_ref[0]

    dot = jax.lax.dot_general(ar, bT, (((1,), (0,)), ((), ())),
                              preferred_element_type=jnp.float32)
    d2 = jnp.maximum(x2c + x2r - 2.0 * dot, 0.0)

    iota = _fiota((R, N), 1)

    def step(k, carry):
        d2, sel, nn_val = carry
        m = jnp.min(d2, axis=1, keepdims=True)
        fi = jnp.min(jnp.where(d2 == m, iota, float(N)), axis=1, keepdims=True)
        onehot_b = iota == fi
        nn_val = jnp.where(k == 1, m, nn_val)
        sel = jnp.where(onehot_b & (k >= 1), 1.0, sel)
        d2 = jnp.where(onehot_b, jnp.inf, d2)
        return d2, sel, nn_val

    init = (d2, jnp.zeros((R, N), jnp.float32), jnp.zeros((R, 1), jnp.float32))
    _, sel, nn_val = jax.lax.fori_loop(0, KNN + 1, step, init)

    ce = [(xT_ref[0, d:d + 1, :] - xr[:, d:d + 1])
          .astype(jnp.bfloat16).astype(jnp.float32) for d in range(3)]
    cc = {}
    for i in range(3):
        for j in range(i, 3):
            cc[(i, j)] = jnp.sum(sel * (ce[i] * ce[j]), axis=1, keepdims=True)
    nn_ref[0] = nn_val
    cov_ref[0] = jnp.concatenate(
        [cc[(min(i, j), max(i, j))] for i in range(3) for j in range(3)], axis=1)


def _knn_pallas(xyz):
    B, N, _ = xyz.shape
    R = 512
    T = N // R
    xb = xyz.astype(jnp.bfloat16)
    xbT = jnp.transpose(xb, (0, 2, 1))
    xT = jnp.transpose(xyz, (0, 2, 1))
    x2 = jnp.sum(xyz * xyz, axis=-1)
    x2c = x2[:, :, None]
    x2r = x2[:, None, :]
    nn, cov9 = pl.pallas_call(
        _knn_kernel,
        grid=(B, T),
        in_specs=[
            pl.BlockSpec((1, R, 3), lambda b, t: (b, t, 0)),
            pl.BlockSpec((1, R, 3), lambda b, t: (b, t, 0)),
            pl.BlockSpec((1, 3, N), lambda b, t: (b, 0, 0)),
            pl.BlockSpec((1, R, 1), lambda b, t: (b, t, 0)),
            pl.BlockSpec((1, 1, N), lambda b, t: (b, 0, 0)),
            pl.BlockSpec((1, 3, N), lambda b, t: (b, 0, 0)),
        ],
        out_specs=[
            pl.BlockSpec((1, R, 1), lambda b, t: (b, t, 0)),
            pl.BlockSpec((1, R, 9), lambda b, t: (b, t, 0)),
        ],
        out_shape=[
            jax.ShapeDtypeStruct((B, N, 1), jnp.float32),
            jax.ShapeDtypeStruct((B, N, 9), jnp.float32),
        ],
    )(xyz, xb, xbT, x2c, x2r, xT)
    return nn[:, :, 0], cov9


def _octree_kernel(scal_ref, gcr_ref, gcc_ref, nrm_ref, pts_ref,
                   nb_ref, ct_ref):
    b = pl.program_id(0)
    N = gcr_ref.shape[2]
    Df = scal_ref[b, 0]
    sinv1 = scal_ref[b, 1]

    gxr = gcr_ref[0, 0:1, :]
    gyr = gcr_ref[0, 1:2, :]
    gzr = gcr_ref[0, 2:3, :]
    gxc = gcc_ref[0, :, 0:1]
    gyc = gcc_ref[0, :, 1:2]
    gzc = gcc_ref[0, :, 2:3]
    nrm = nrm_ref[0]
    pcol = pts_ref[0]
    feat = jnp.concatenate([nrm, jnp.ones((N, 1), jnp.float32)], axis=1)

    slots = _fiota((1, CP), 1)
    c8col = _fiota((8, 1), 0)
    c8row = _fiota((1, 8), 1)
    bx8 = jnp.floor(c8col * 0.25)
    by8 = jnp.floor(c8col * 0.5) - 2.0 * bx8
    bz8 = c8col - 2.0 * jnp.floor(c8col * 0.5)
    i8 = _fiota((8, 8), 0)
    j8 = _fiota((8, 8), 1)
    tri8 = (j8 < i8).astype(jnp.float32)

    def child_stats(pm_row, pm_col, sinv_c):
        def bits(gr):
            return jnp.floor(gr * sinv_c) - 2.0 * jnp.floor(gr * (0.5 * sinv_c))
        ccr = 4.0 * bits(gxr) + 2.0 * bits(gyr) + bits(gzr)
        ccc = 4.0 * bits(gxc) + 2.0 * bits(gyc) + bits(gzc)
        oh = jnp.where((ccr == c8col) & pm_row, 1.0, 0.0)
        ohT = jnp.where((ccc == c8row) & pm_col, 1.0, 0.0)
        sums = jax.lax.dot_general(oh, feat, (((1,), (0,)), ((), ())),
                                   precision=HI,
                                   preferred_element_type=jnp.float32)
        cnt = sums[:, 3:4]
        safe = jnp.maximum(cnt, 1.0)
        mean = sums[:, 0:3] / safe
        meanpt = jax.lax.dot_general(ohT, mean, (((1,), (0,)), ((), ())),
                                     precision=HI,
                                     preferred_element_type=jnp.float32)
        diff = nrm - meanpt
        sq = (diff[:, 0:1] * diff[:, 0:1] + diff[:, 1:2] * diff[:, 1:2]) \
            + diff[:, 2:3] * diff[:, 2:3]
        vsum = jax.lax.dot_general(oh, sq, (((1,), (0,)), ((), ())),
                                   precision=HI,
                                   preferred_element_type=jnp.float32)
        var = jnp.where(cnt >= float(MIN_PTS), vsum / safe, 0.0)
        return cnt, var

    def insert(state_sh, vals, pos, validf):
        match = jnp.where((slots == pos) & (validf > 0.0), 1.0, 0.0)
        add = jnp.sum(vals * match, axis=0, keepdims=True)
        anym = jnp.max(match, axis=0, keepdims=True)
        return state_sh * (1.0 - anym) + add

    ones_row = jnp.ones((1, N), jnp.bool_)
    ones_col = jnp.ones((N, 1), jnp.bool_)
    cnt0, var0 = child_stats(ones_row, ones_col, sinv1)
    valid0 = (cnt0 >= 1.0).astype(jnp.float32)
    spl0 = valid0 * ((1.0 < Df) & (cnt0 >= float(MIN_PTS))).astype(jnp.float32)
    rank0 = jax.lax.dot_general(tri8, valid0, (((1,), (0,)), ((), ())),
                                precision=HI, preferred_element_type=jnp.float32)
    n0 = jnp.sum(valid0)
    zero = jnp.zeros((1, CP), jnp.float32)
    kx = insert(zero, bx8, rank0, valid0)
    ky = insert(zero, by8, rank0, valid0)
    kz = insert(zero, bz8, rank0, valid0)
    dd = insert(jnp.full((1, CP), 1.0), jnp.ones((8, 1), jnp.float32), rank0, valid0)
    var = insert(zero, var0, rank0, valid0)
    cnt = insert(zero, cnt0, rank0, valid0)
    spl = insert(zero, spl0, rank0, valid0)
    sinv = insert(jnp.full((1, CP), sinv1), jnp.full((8, 1), sinv1), rank0, valid0)
    cont0 = (n0 < float(NG)) & (jnp.max(jnp.where(slots < n0, spl, 0.0)) > 0.0)

    def cond(carry):
        return carry[-1]

    def body(carry):
        kx, ky, kz, dd, var, cnt, spl, sinv, n, _ = carry
        active = slots < n
        scores = jnp.where(active & (spl > 0.0), var, -jnp.inf)
        smax = jnp.max(scores)
        si = jnp.min(jnp.where(scores == smax, slots, float(CP)))
        sel1 = jnp.where(slots == si, 1.0, 0.0)
        pd = jnp.sum(sel1 * dd)
        kxp = jnp.sum(sel1 * kx)
        kyp = jnp.sum(sel1 * ky)
        kzp = jnp.sum(sel1 * kz)
        sinv_p = jnp.sum(sel1 * sinv)

        pm_row = (jnp.floor(gxr * sinv_p) == kxp) \
            & (jnp.floor(gyr * sinv_p) == kyp) \
            & (jnp.floor(gzr * sinv_p) == kzp)
        pm_col = (jnp.floor(gxc * sinv_p) == kxp) \
            & (jnp.floor(gyc * sinv_p) == kyp) \
            & (jnp.floor(gzc * sinv_p) == kzp)
        sinv_c = 2.0 * sinv_p
        cnt_c, var_c = child_stats(pm_row, pm_col, sinv_c)
        ckx = 2.0 * kxp + bx8
        cky = 2.0 * kyp + by8
        ckz = 2.0 * kzp + bz8
        cdd = pd + 1.0
        validf = (cnt_c >= 1.0).astype(jnp.float32)
        csplf = validf * ((cdd < Df) & (cnt_c >= float(MIN_PTS))).astype(jnp.float32)
        rankv = jax.lax.dot_general(tri8, validf, (((1,), (0,)), ((), ())),
                                    precision=HI,
                                    preferred_element_type=jnp.float32)
        nch = jnp.sum(validf)
        keep = n - 1.0
        pos = keep + rankv

        def shift(S):
            return jnp.where(slots < si, S,
                             jnp.concatenate([S[:, 1:], S[:, -1:]], axis=1))

        e8 = jnp.ones((8, 1), jnp.float32)
        kx = insert(shift(kx), ckx, pos, validf)
        ky = insert(shift(ky), cky, pos, validf)
        kz = insert(shift(kz), ckz, pos, validf)
        dd = insert(shift(dd), cdd * e8, pos, validf)
        var = insert(shift(var), var_c, pos, validf)
        cnt = insert(shift(cnt), cnt_c, pos, validf)
        spl = insert(shift(spl), csplf, pos, validf)
        sinv = insert(shift(sinv), sinv_c * e8, pos, validf)
        n = keep + nch
        cont = (n < float(NG)) & (jnp.max(jnp.where(slots < n, spl, 0.0)) > 0.0)
        return kx, ky, kz, dd, var, cnt, spl, sinv, n, cont

    carry = jax.lax.while_loop(
        cond, body, (kx, ky, kz, dd, var, cnt, spl, sinv, n0, cont0))
    kx, ky, kz, dd, var, cnt, spl, sinv, n, _ = carry

    active = slots < n
    iCP = _fiota((CP, CP), 0)
    jCP = _fiota((CP, CP), 1)

    def stable_order(key_row):
        kc = jnp.sum(jnp.where(iCP == jCP, key_row, 0.0), axis=1,
                     keepdims=True)
        cmp = (key_row < kc) | ((key_row == kc) & (jCP < iCP))
        rank = jnp.sum(cmp.astype(jnp.float32), axis=1, keepdims=True)
        onehot = (rank == jCP).astype(jnp.float32)
        icol = _fiota((CP, 1), 0)
        return jnp.sum(icol * onehot, axis=0, keepdims=True)

    inf = jnp.inf
    orda = stable_order(jnp.where(active, var, inf))
    ordd = stable_order(jnp.where(active, -var, inf))

    j128 = _fiota((NG, 1), 0)
    p128 = _fiota((NG, CP), 1)

    def gather_row(row, idx):
        m = (p128 == idx).astype(jnp.float32)
        return jnp.sum(m * row, axis=1, keepdims=True)

    start = jnp.clip(n - float(NG), 0.0, float(CP - NG))
    start = jnp.minimum(start, 8.0)
    selA = gather_row(orda, start + j128)
    selB = j128
    n_safe = jnp.maximum(n, 1.0)
    jm = j128 - n
    jw = jnp.where(j128 < n, j128, jm - n_safe * jnp.floor(jm / n_safe))
    selC = gather_row(ordd, jw)
    sel = jnp.where(n > float(NG), selA, jnp.where(n == float(NG), selB, selC))

    kxg = gather_row(kx, sel)
    kyg = gather_row(ky, sel)
    kzg = gather_row(kz, sel)
    cntg = gather_row(cnt, sel)
    sinvg = gather_row(sinv, sel)

    maskb = (jnp.floor(gxr * sinvg) == kxg) \
        & (jnp.floor(gyr * sinvg) == kyg) \
        & (jnp.floor(gzr * sinvg) == kzg)
    maskf = maskb.astype(jnp.float32)

    ctr = jax.lax.dot_general(maskf, pcol, (((1,), (0,)), ((), ())),
                              precision=HI,
                              preferred_element_type=jnp.float32) / cntg

    r = maskf
    s = 1
    while s < N:
        z = jnp.zeros((NG, s), jnp.float32)
        r = r + jnp.concatenate([z, r[:, :-s]], axis=1)
        s *= 2
    rank0b = r - 1.0

    pieces = []
    scale = jnp.full((NG, 1), -jnp.inf)
    for t in range(GS):
        tf = float(t)
        tmod = tf - cntg * jnp.floor(tf / cntg)
        hit = ((rank0b == tmod) & maskb).astype(jnp.float32)
        pt = jax.lax.dot_general(hit, pcol, (((1,), (0,)), ((), ())),
                                 precision=HI,
                                 preferred_element_type=jnp.float32)
        nbt = pt - ctr
        nsq = (nbt[:, 0:1] * nbt[:, 0:1] + nbt[:, 1:2] * nbt[:, 1:2]) \
            + nbt[:, 2:3] * nbt[:, 2:3]
        scale = jnp.maximum(scale, jnp.sqrt(nsq + 1e-12))
        pieces.append(nbt)
    scale = jnp.maximum(scale, 1e-6)
    nb_ref[0] = jnp.concatenate(pieces, axis=1) / scale
    ct_ref[0] = ctr


def _octree_pallas(scal, gcr, gcc, nrm, pts):
    B, _, N = gcr.shape
    grid_spec = pltpu.PrefetchScalarGridSpec(
        num_scalar_prefetch=1,
        grid=(B,),
        in_specs=[
            pl.BlockSpec((1, 3, N), lambda b, s: (b, 0, 0)),
            pl.BlockSpec((1, N, 3), lambda b, s: (b, 0, 0)),
            pl.BlockSpec((1, N, 3), lambda b, s: (b, 0, 0)),
            pl.BlockSpec((1, N, 3), lambda b, s: (b, 0, 0)),
        ],
        out_specs=[
            pl.BlockSpec((1, NG, 3 * GS), lambda b, s: (b, 0, 0)),
            pl.BlockSpec((1, NG, 3), lambda b, s: (b, 0, 0)),
        ],
    )
    nb, ct = pl.pallas_call(
        _octree_kernel,
        grid_spec=grid_spec,
        out_shape=[
            jax.ShapeDtypeStruct((B, NG, 3 * GS), jnp.float32),
            jax.ShapeDtypeStruct((B, NG, 3), jnp.float32),
        ],
    )(scal, gcr, gcc, nrm, pts)
    return nb, ct


_POW2 = np.ldexp(1.0, 1 - np.arange(32)).astype(np.float32)


def kernel(xyz):
    B, N, _ = xyz.shape
    nn_d2, cov9 = _knn_pallas(xyz)
    cov = cov9.reshape(B, N, 3, 3) / float(KNN)
    _, eigvecs = jnp.linalg.eigh(cov)
    normals = eigvecs[..., :, 0]
    nn_dists = jnp.sqrt(jnp.maximum(nn_d2, 1e-08))

    grid_size = jnp.maximum(jnp.quantile(nn_dists, 0.5, axis=1), 1e-06)
    xyz_min = xyz.min(axis=1)
    xyz_max = xyz.max(axis=1)
    bbox_diag = jnp.linalg.norm(xyz_max - xyz_min, axis=-1)
    raw_depth = jnp.log2(jnp.maximum(bbox_diag / grid_size, 1.0))
    depth = jnp.clip(jnp.ceil(raw_depth).astype(jnp.int32), 8, 16)
    gcf = jnp.floor((xyz - xyz_min[:, None, :]) / grid_size[:, None, None])

    scal = jnp.stack([depth.astype(jnp.float32),
                      jnp.take(jnp.asarray(_POW2), depth)], axis=1)
    gcr = jnp.transpose(gcf, (0, 2, 1))
    nb, ct = _octree_pallas(scal, gcr, gcf, normals, xyz)
    return nb.reshape(B, NG, GS, 3), ct


# speedup vs baseline: 1.3687x; 1.0289x over previous
import functools

import jax
import jax.numpy as jnp
import numpy as np
from jax.experimental import pallas as pl
from jax.experimental.pallas import tpu as pltpu

NG = 128
GS = 32
KNN = 16
MIN_PTS = 4
CP = 256
HI = jax.lax.Precision.HIGHEST


def _fiota(shape, dim):
    return jax.lax.broadcasted_iota(jnp.int32, shape, dim).astype(jnp.float32)


def _knn_kernel(rows_ref, br_ref, bT_ref, x2c_ref, x2r_ref, xT_ref,
                nn_ref, cov_ref):
    R = rows_ref.shape[1]
    N = bT_ref.shape[2]
    xr = rows_ref[0]
    ar = br_ref[0]
    bT = bT_ref[0]
    x2c = x2c_ref[0]
    x2r = x2r_ref[0]

    dot = jax.lax.dot_general(ar, bT, (((1,), (0,)), ((), ())),
                              preferred_element_type=jnp.float32)
    d2 = jnp.maximum(x2c + x2r - 2.0 * dot, 0.0)

    iota = _fiota((R, N), 1)

    def step(k, carry):
        d2, fi0, nn_val = carry
        m = jnp.min(d2, axis=1, keepdims=True)
        fi = jnp.min(jnp.where(d2 == m, iota, float(N)), axis=1, keepdims=True)
        onehot_b = iota == fi
        nn_val = jnp.where(k == 1, m, nn_val)
        fi0 = jnp.where(k == 0, fi, fi0)
        d2 = jnp.where(onehot_b, jnp.inf, d2)
        return d2, fi0, nn_val

    init = (d2, jnp.zeros((R, 1), jnp.float32), jnp.zeros((R, 1), jnp.float32))
    d2, fi0, nn_val = jax.lax.fori_loop(0, KNN + 1, step, init)
    sel = ((d2 == jnp.inf) & (iota != fi0)).astype(jnp.float32)

    ce = [(xT_ref[0, d:d + 1, :] - xr[:, d:d + 1])
          .astype(jnp.bfloat16).astype(jnp.float32) for d in range(3)]
    cc = {}
    for i in range(3):
        for j in range(i, 3):
            cc[(i, j)] = jnp.sum(sel * (ce[i] * ce[j]), axis=1, keepdims=True)
    nn_ref[0] = nn_val
    cov_ref[0] = jnp.concatenate(
        [cc[(min(i, j), max(i, j))] for i in range(3) for j in range(3)], axis=1)


def _knn_pallas(xyz):
    B, N, _ = xyz.shape
    R = 512
    T = N // R
    xb = xyz.astype(jnp.bfloat16)
    xbT = jnp.transpose(xb, (0, 2, 1))
    xT = jnp.transpose(xyz, (0, 2, 1))
    x2 = jnp.sum(xyz * xyz, axis=-1)
    x2c = x2[:, :, None]
    x2r = x2[:, None, :]
    nn, cov9 = pl.pallas_call(
        _knn_kernel,
        grid=(B, T),
        in_specs=[
            pl.BlockSpec((1, R, 3), lambda b, t: (b, t, 0)),
            pl.BlockSpec((1, R, 3), lambda b, t: (b, t, 0)),
            pl.BlockSpec((1, 3, N), lambda b, t: (b, 0, 0)),
            pl.BlockSpec((1, R, 1), lambda b, t: (b, t, 0)),
            pl.BlockSpec((1, 1, N), lambda b, t: (b, 0, 0)),
            pl.BlockSpec((1, 3, N), lambda b, t: (b, 0, 0)),
        ],
        out_specs=[
            pl.BlockSpec((1, R, 1), lambda b, t: (b, t, 0)),
            pl.BlockSpec((1, R, 9), lambda b, t: (b, t, 0)),
        ],
        out_shape=[
            jax.ShapeDtypeStruct((B, N, 1), jnp.float32),
            jax.ShapeDtypeStruct((B, N, 9), jnp.float32),
        ],
    )(xyz, xb, xbT, x2c, x2r, xT)
    return nn[:, :, 0], cov9


def _octree_kernel(scal_ref, gcr_ref, gcc_ref, nrm_ref, pts_ref,
                   nb_ref, ct_ref):
    b = pl.program_id(0)
    N = gcr_ref.shape[2]
    Df = scal_ref[b, 0]
    sinv1 = scal_ref[b, 1]

    gxr = gcr_ref[0, 0:1, :]
    gyr = gcr_ref[0, 1:2, :]
    gzr = gcr_ref[0, 2:3, :]
    gxc = gcc_ref[0, :, 0:1]
    gyc = gcc_ref[0, :, 1:2]
    gzc = gcc_ref[0, :, 2:3]
    nrm = nrm_ref[0]
    pcol = pts_ref[0]
    feat = jnp.concatenate([nrm, jnp.ones((N, 1), jnp.float32)], axis=1)

    slots = _fiota((1, CP), 1)
    c8col = _fiota((8, 1), 0)
    c8row = _fiota((1, 8), 1)
    bx8 = jnp.floor(c8col * 0.25)
    by8 = jnp.floor(c8col * 0.5) - 2.0 * bx8
    bz8 = c8col - 2.0 * jnp.floor(c8col * 0.5)
    i8 = _fiota((8, 8), 0)
    j8 = _fiota((8, 8), 1)
    tri8 = (j8 < i8).astype(jnp.float32)

    def child_stats(pm_row, pm_col, sinv_c):
        def bits(gr):
            return jnp.floor(gr * sinv_c) - 2.0 * jnp.floor(gr * (0.5 * sinv_c))
        ccr = 4.0 * bits(gxr) + 2.0 * bits(gyr) + bits(gzr)
        ccc = 4.0 * bits(gxc) + 2.0 * bits(gyc) + bits(gzc)
        oh = jnp.where((ccr == c8col) & pm_row, 1.0, 0.0)
        ohT = jnp.where((ccc == c8row) & pm_col, 1.0, 0.0)
        sums = jax.lax.dot_general(oh, feat, (((1,), (0,)), ((), ())),
                                   precision=HI,
                                   preferred_element_type=jnp.float32)
        cnt = sums[:, 3:4]
        safe = jnp.maximum(cnt, 1.0)
        mean = sums[:, 0:3] / safe
        meanpt = jax.lax.dot_general(ohT, mean, (((1,), (0,)), ((), ())),
                                     precision=HI,
                                     preferred_element_type=jnp.float32)
        diff = nrm - meanpt
        sq = (diff[:, 0:1] * diff[:, 0:1] + diff[:, 1:2] * diff[:, 1:2]) \
            + diff[:, 2:3] * diff[:, 2:3]
        vsum = jax.lax.dot_general(oh, sq, (((1,), (0,)), ((), ())),
                                   precision=HI,
                                   preferred_element_type=jnp.float32)
        var = jnp.where(cnt >= float(MIN_PTS), vsum / safe, 0.0)
        return cnt, var

    def insert(state_sh, vals, pos, validf):
        match = jnp.where((slots == pos) & (validf > 0.0), 1.0, 0.0)
        add = jnp.sum(vals * match, axis=0, keepdims=True)
        anym = jnp.max(match, axis=0, keepdims=True)
        return state_sh * (1.0 - anym) + add

    ones_row = jnp.ones((1, N), jnp.bool_)
    ones_col = jnp.ones((N, 1), jnp.bool_)
    cnt0, var0 = child_stats(ones_row, ones_col, sinv1)
    valid0 = (cnt0 >= 1.0).astype(jnp.float32)
    spl0 = valid0 * ((1.0 < Df) & (cnt0 >= float(MIN_PTS))).astype(jnp.float32)
    rank0 = jax.lax.dot_general(tri8, valid0, (((1,), (0,)), ((), ())),
                                precision=HI, preferred_element_type=jnp.float32)
    n0 = jnp.sum(valid0)
    zero = jnp.zeros((1, CP), jnp.float32)
    kx = insert(zero, bx8, rank0, valid0)
    ky = insert(zero, by8, rank0, valid0)
    kz = insert(zero, bz8, rank0, valid0)
    dd = insert(jnp.full((1, CP), 1.0), jnp.ones((8, 1), jnp.float32), rank0, valid0)
    var = insert(zero, var0, rank0, valid0)
    cnt = insert(zero, cnt0, rank0, valid0)
    spl = insert(zero, spl0, rank0, valid0)
    sinv = insert(jnp.full((1, CP), sinv1), jnp.full((8, 1), sinv1), rank0, valid0)
    cont0 = (n0 < float(NG)) & (jnp.max(jnp.where(slots < n0, spl, 0.0)) > 0.0)

    def cond(carry):
        return carry[-1]

    def body(carry):
        kx, ky, kz, dd, var, cnt, spl, sinv, n, _ = carry
        active = slots < n
        scores = jnp.where(active & (spl > 0.0), var, -jnp.inf)
        smax = jnp.max(scores)
        si = jnp.min(jnp.where(scores == smax, slots, float(CP)))
        sel1 = jnp.where(slots == si, 1.0, 0.0)
        pd = jnp.sum(sel1 * dd)
        kxp = jnp.sum(sel1 * kx)
        kyp = jnp.sum(sel1 * ky)
        kzp = jnp.sum(sel1 * kz)
        sinv_p = jnp.sum(sel1 * sinv)

        pm_row = (jnp.floor(gxr * sinv_p) == kxp) \
            & (jnp.floor(gyr * sinv_p) == kyp) \
            & (jnp.floor(gzr * sinv_p) == kzp)
        pm_col = (jnp.floor(gxc * sinv_p) == kxp) \
            & (jnp.floor(gyc * sinv_p) == kyp) \
            & (jnp.floor(gzc * sinv_p) == kzp)
        sinv_c = 2.0 * sinv_p
        cnt_c, var_c = child_stats(pm_row, pm_col, sinv_c)
        ckx = 2.0 * kxp + bx8
        cky = 2.0 * kyp + by8
        ckz = 2.0 * kzp + bz8
        cdd = pd + 1.0
        validf = (cnt_c >= 1.0).astype(jnp.float32)
        csplf = validf * ((cdd < Df) & (cnt_c >= float(MIN_PTS))).astype(jnp.float32)
        rankv = jax.lax.dot_general(tri8, validf, (((1,), (0,)), ((), ())),
                                    precision=HI,
                                    preferred_element_type=jnp.float32)
        nch = jnp.sum(validf)
        keep = n - 1.0
        pos = keep + rankv

        def shift(S):
            return jnp.where(slots < si, S,
                             jnp.concatenate([S[:, 1:], S[:, -1:]], axis=1))

        e8 = jnp.ones((8, 1), jnp.float32)
        kx = insert(shift(kx), ckx, pos, validf)
        ky = insert(shift(ky), cky, pos, validf)
        kz = insert(shift(kz), ckz, pos, validf)
        dd = insert(shift(dd), cdd * e8, pos, validf)
        var = insert(shift(var), var_c, pos, validf)
        cnt = insert(shift(cnt), cnt_c, pos, validf)
        spl = insert(shift(spl), csplf, pos, validf)
        sinv = insert(shift(sinv), sinv_c * e8, pos, validf)
        n = keep + nch
        cont = (n < float(NG)) & (jnp.max(jnp.where(slots < n, spl, 0.0)) > 0.0)
        return kx, ky, kz, dd, var, cnt, spl, sinv, n, cont

    carry = jax.lax.while_loop(
        cond, body, (kx, ky, kz, dd, var, cnt, spl, sinv, n0, cont0))
    kx, ky, kz, dd, var, cnt, spl, sinv, n, _ = carry

    active = slots < n
    iCP = _fiota((CP, CP), 0)
    jCP = _fiota((CP, CP), 1)

    def stable_order(key_row):
        kc = jnp.sum(jnp.where(iCP == jCP, key_row, 0.0), axis=1,
                     keepdims=True)
        cmp = (key_row < kc) | ((key_row == kc) & (jCP < iCP))
        rank = jnp.sum(cmp.astype(jnp.float32), axis=1, keepdims=True)
        onehot = (rank == jCP).astype(jnp.float32)
        icol = _fiota((CP, 1), 0)
        return jnp.sum(icol * onehot, axis=0, keepdims=True)

    inf = jnp.inf
    orda = stable_order(jnp.where(active, var, inf))
    ordd = stable_order(jnp.where(active, -var, inf))

    j128 = _fiota((NG, 1), 0)
    p128 = _fiota((NG, CP), 1)

    def gather_row(row, idx):
        m = (p128 == idx).astype(jnp.float32)
        return jnp.sum(m * row, axis=1, keepdims=True)

    start = jnp.clip(n - float(NG), 0.0, float(CP - NG))
    start = jnp.minimum(start, 8.0)
    selA = gather_row(orda, start + j128)
    selB = j128
    n_safe = jnp.maximum(n, 1.0)
    jm = j128 - n
    jw = jnp.where(j128 < n, j128, jm - n_safe * jnp.floor(jm / n_safe))
    selC = gather_row(ordd, jw)
    sel = jnp.where(n > float(NG), selA, jnp.where(n == float(NG), selB, selC))

    kxg = gather_row(kx, sel)
    kyg = gather_row(ky, sel)
    kzg = gather_row(kz, sel)
    cntg = gather_row(cnt, sel)
    sinvg = gather_row(sinv, sel)

    maskb = (jnp.floor(gxr * sinvg) == kxg) \
        & (jnp.floor(gyr * sinvg) == kyg) \
        & (jnp.floor(gzr * sinvg) == kzg)
    maskf = maskb.astype(jnp.float32)

    ctr = jax.lax.dot_general(maskf, pcol, (((1,), (0,)), ((), ())),
                              precision=HI,
                              preferred_element_type=jnp.float32) / cntg

    r = maskf
    s = 1
    while s < N:
        z = jnp.zeros((NG, s), jnp.float32)
        r = r + jnp.concatenate([z, r[:, :-s]], axis=1)
        s *= 2
    rank0b = r - 1.0

    pieces = []
    scale = jnp.full((NG, 1), -jnp.inf)
    for t in range(GS):
        tf = float(t)
        tmod = tf - cntg * jnp.floor(tf / cntg)
        hit = ((rank0b == tmod) & maskb).astype(jnp.float32)
        pt = jax.lax.dot_general(hit, pcol, (((1,), (0,)), ((), ())),
                                 precision=HI,
                                 preferred_element_type=jnp.float32)
        nbt = pt - ctr
        nsq = (nbt[:, 0:1] * nbt[:, 0:1] + nbt[:, 1:2] * nbt[:, 1:2]) \
            + nbt[:, 2:3] * nbt[:, 2:3]
        scale = jnp.maximum(scale, jnp.sqrt(nsq + 1e-12))
        pieces.append(nbt)
    scale = jnp.maximum(scale, 1e-6)
    nb_ref[0] = jnp.concatenate(pieces, axis=1) / scale
    ct_ref[0] = ctr


def _octree_pallas(scal, gcr, gcc, nrm, pts):
    B, _, N = gcr.shape
    grid_spec = pltpu.PrefetchScalarGridSpec(
        num_scalar_prefetch=1,
        grid=(B,),
        in_specs=[
            pl.BlockSpec((1, 3, N), lambda b, s: (b, 0, 0)),
            pl.BlockSpec((1, N, 3), lambda b, s: (b, 0, 0)),
            pl.BlockSpec((1, N, 3), lambda b, s: (b, 0, 0)),
            pl.BlockSpec((1, N, 3), lambda b, s: (b, 0, 0)),
        ],
        out_specs=[
            pl.BlockSpec((1, NG, 3 * GS), lambda b, s: (b, 0, 0)),
            pl.BlockSpec((1, NG, 3), lambda b, s: (b, 0, 0)),
        ],
    )
    nb, ct = pl.pallas_call(
        _octree_kernel,
        grid_spec=grid_spec,
        out_shape=[
            jax.ShapeDtypeStruct((B, NG, 3 * GS), jnp.float32),
            jax.ShapeDtypeStruct((B, NG, 3), jnp.float32),
        ],
    )(scal, gcr, gcc, nrm, pts)
    return nb, ct


_POW2 = np.ldexp(1.0, 1 - np.arange(32)).astype(np.float32)


def kernel(xyz):
    B, N, _ = xyz.shape
    nn_d2, cov9 = _knn_pallas(xyz)
    cov = cov9.reshape(B, N, 3, 3) / float(KNN)
    _, eigvecs = jnp.linalg.eigh(cov)
    normals = eigvecs[..., :, 0]
    nn_dists = jnp.sqrt(jnp.maximum(nn_d2, 1e-08))

    grid_size = jnp.maximum(jnp.quantile(nn_dists, 0.5, axis=1), 1e-06)
    xyz_min = xyz.min(axis=1)
    xyz_max = xyz.max(axis=1)
    bbox_diag = jnp.linalg.norm(xyz_max - xyz_min, axis=-1)
    raw_depth = jnp.log2(jnp.maximum(bbox_diag / grid_size, 1.0))
    depth = jnp.clip(jnp.ceil(raw_depth).astype(jnp.int32), 8, 16)
    gcf = jnp.floor((xyz - xyz_min[:, None, :]) / grid_size[:, None, None])

    scal = jnp.stack([depth.astype(jnp.float32),
                      jnp.take(jnp.asarray(_POW2), depth)], axis=1)
    gcr = jnp.transpose(gcf, (0, 2, 1))
    nb, ct = _octree_pallas(scal, gcr, gcf, normals, xyz)
    return nb.reshape(B, NG, GS, 3), ct
